# baseline (device time: 32600 ns/iter reference)
import jax
import jax.numpy as jnp
from jax import lax
from jax.experimental import pallas as pl
from jax.experimental.pallas import tpu as pltpu

N_DEV = 32
NQ = 4
KC = 4
SLOTS = 8
WINDOW = 4


def kernel(x, w_mat, scale_x, scale_w):
    m_per, k = x.shape
    n = w_mat.shape[1]
    n_per = n // N_DEV
    qn = n // NQ
    qk = k // KC
    nchunk = NQ * KC
    dpq = N_DEV // NQ

    def body(x_ref, w_ref, sx_ref, sw_ref, out_ref,
             wbufs, zbuf, recv_buf, load_sems, send_sems, recv_sems,
             copy_sem):
        my = lax.axis_index("i")

        def chunk_src(idx):
            nq_, kc_ = idx // KC, idx % KC
            return w_ref.at[pl.ds(kc_ * qk, qk), pl.ds(nq_ * qn, qn)]

        def start_load(idx):
            cp = pltpu.make_async_copy(
                chunk_src(idx), wbufs.at[idx % SLOTS],
                load_sems.at[idx % SLOTS])
            cp.start()
            return cp

        loads = {}
        for idx in range(min(WINDOW, nchunk)):
            loads[idx] = start_load(idx)

        barrier = pltpu.get_barrier_semaphore()
        for h in range(1, N_DEV):
            pl.semaphore_signal(
                barrier, inc=1,
                device_id=((my + h) % N_DEV,),
                device_id_type=pl.DeviceIdType.MESH,
            )
        pl.semaphore_wait(barrier, N_DEV - 1)

        xb = x_ref[...].astype(jnp.bfloat16)
        scale = sx_ref[0] * sw_ref[0]

        acc = None
        for idx in range(nchunk):
            nq_, kc_ = idx // KC, idx % KC
            loads[idx].wait()
            if idx + WINDOW < nchunk:
                loads[idx + WINDOW] = start_load(idx + WINDOW)
            wb = wbufs[idx % SLOTS].astype(jnp.bfloat16)
            part = lax.dot_general(
                xb[:, kc_ * qk:(kc_ + 1) * qk], wb,
                (((1,), (0,)), ((), ())),
                preferred_element_type=jnp.float32)
            acc = part if kc_ == 0 else acc + part
            if kc_ == KC - 1:
                y = acc * scale
                z = (y * jax.nn.sigmoid(y)).astype(jnp.bfloat16)
                for j in range(dpq):
                    zbuf[nq_ * dpq + j] = z[:, j * n_per:(j + 1) * n_per]
                for j in range(dpq):
                    dst = nq_ * dpq + lax.rem(my + j, dpq)

                    @pl.when(dst != my)
                    def _(dst=dst):
                        rdma = pltpu.make_async_remote_copy(
                            src_ref=zbuf.at[dst],
                            dst_ref=recv_buf.at[my],
                            send_sem=send_sems.at[dst],
                            recv_sem=recv_sems.at[my],
                            device_id=(dst,),
                            device_id_type=pl.DeviceIdType.MESH,
                        )
                        rdma.start()

                    @pl.when(dst == my)
                    def _(dst=dst):
                        local = pltpu.make_async_copy(
                            zbuf.at[my], recv_buf.at[my], copy_sem)
                        local.start()
                        local.wait()

        for h in range(1, N_DEV):
            src = (my - h) % N_DEV
            pltpu.make_async_remote_copy(
                src_ref=zbuf.at[0],
                dst_ref=recv_buf.at[src],
                send_sem=send_sems.at[0],
                recv_sem=recv_sems.at[src],
                device_id=(src,),
                device_id_type=pl.DeviceIdType.MESH,
            ).wait_recv()
        for s in range(N_DEV):

            @pl.when(s != my)
            def _(s=s):
                pltpu.make_async_remote_copy(
                    src_ref=zbuf.at[s],
                    dst_ref=recv_buf.at[my],
                    send_sem=send_sems.at[s],
                    recv_sem=recv_sems.at[my],
                    device_id=((my + 1) % N_DEV,),
                    device_id_type=pl.DeviceIdType.MESH,
                ).wait_send()

        out_ref[...] = recv_buf[...].astype(jnp.float32).reshape(
            N_DEV * m_per, n_per)

    return pl.pallas_call(
        body,
        out_shape=jax.ShapeDtypeStruct((N_DEV * m_per, n_per), jnp.float32),
        in_specs=[
            pl.BlockSpec(memory_space=pltpu.VMEM),
            pl.BlockSpec(memory_space=pltpu.MemorySpace.HBM),
            pl.BlockSpec(memory_space=pltpu.SMEM),
            pl.BlockSpec(memory_space=pltpu.SMEM),
        ],
        out_specs=pl.BlockSpec(memory_space=pltpu.VMEM),
        scratch_shapes=[
            pltpu.VMEM((SLOTS, k // KC, n // NQ), jnp.float32),
            pltpu.VMEM((N_DEV, m_per, n // N_DEV), jnp.bfloat16),
            pltpu.VMEM((N_DEV, m_per, n // N_DEV), jnp.bfloat16),
            pltpu.SemaphoreType.DMA((SLOTS,)),
            pltpu.SemaphoreType.DMA((N_DEV,)),
            pltpu.SemaphoreType.DMA((N_DEV,)),
            pltpu.SemaphoreType.DMA,
        ],
        compiler_params=pltpu.CompilerParams(
            collective_id=0,
            vmem_limit_bytes=128 * 1024 * 1024,
        ),
    )(x, w_mat, scale_x, scale_w)


# device time: 28958 ns/iter; 1.1258x vs baseline; 1.1258x over previous
import jax
import jax.numpy as jnp
from jax import lax
from jax.experimental import pallas as pl
from jax.experimental.pallas import tpu as pltpu

N_DEV = 32
NQ = 4
KC = 4
SLOTS = 8
WINDOW = 6


def kernel(x, w_mat, scale_x, scale_w):
    m_per, k = x.shape
    n = w_mat.shape[1]
    n_per = n // N_DEV
    qn = n // NQ
    qk = k // KC
    nchunk = NQ * KC
    dpq = N_DEV // NQ

    def body(x_ref, w_ref, sx_ref, sw_ref, out_ref,
             wbufs, zbuf, recv_buf, load_sems, send_sems, recv_sems,
             copy_sem):
        my = lax.axis_index("i")

        def chunk_src(idx):
            nq_, kc_ = idx // KC, idx % KC
            return w_ref.at[pl.ds(kc_ * qk, qk), pl.ds(nq_ * qn, qn)]

        def start_load(idx):
            cp = pltpu.make_async_copy(
                chunk_src(idx), wbufs.at[idx % SLOTS],
                load_sems.at[idx % SLOTS])
            cp.start()
            return cp

        loads = {}
        for idx in range(min(WINDOW, nchunk)):
            loads[idx] = start_load(idx)

        barrier = pltpu.get_barrier_semaphore()
        for h in range(1, N_DEV):
            pl.semaphore_signal(
                barrier, inc=1,
                device_id=((my + h) % N_DEV,),
                device_id_type=pl.DeviceIdType.MESH,
            )

        xb = x_ref[...].astype(jnp.bfloat16)
        scale = sx_ref[0] * sw_ref[0]

        acc = None
        for idx in range(nchunk):
            nq_, kc_ = idx // KC, idx % KC
            loads[idx].wait()
            if idx + WINDOW < nchunk:
                loads[idx + WINDOW] = start_load(idx + WINDOW)
            wb = wbufs[idx % SLOTS].astype(jnp.bfloat16)
            part = lax.dot_general(
                xb[:, kc_ * qk:(kc_ + 1) * qk], wb,
                (((1,), (0,)), ((), ())),
                preferred_element_type=jnp.float32)
            acc = part if kc_ == 0 else acc + part
            if kc_ == KC - 1:
                y = acc * scale
                z = (y * jax.nn.sigmoid(y)).astype(jnp.bfloat16)
                for j in range(dpq):
                    zbuf[nq_ * dpq + j] = z[:, j * n_per:(j + 1) * n_per]
                if nq_ == 0:
                    pl.semaphore_wait(barrier, N_DEV - 1)
                for j in range(dpq):
                    dst = nq_ * dpq + lax.rem(my + j, dpq)

                    @pl.when(dst != my)
                    def _(dst=dst):
                        rdma = pltpu.make_async_remote_copy(
                            src_ref=zbuf.at[dst],
                            dst_ref=recv_buf.at[my],
                            send_sem=send_sems.at[dst],
                            recv_sem=recv_sems.at[my],
                            device_id=(dst,),
                            device_id_type=pl.DeviceIdType.MESH,
                        )
                        rdma.start()

                    @pl.when(dst == my)
                    def _(dst=dst):
                        local = pltpu.make_async_copy(
                            zbuf.at[my], recv_buf.at[my], copy_sem)
                        local.start()
                        local.wait()

        for h in range(1, N_DEV):
            src = (my - h) % N_DEV
            pltpu.make_async_remote_copy(
                src_ref=zbuf.at[0],
                dst_ref=recv_buf.at[src],
                send_sem=send_sems.at[0],
                recv_sem=recv_sems.at[src],
                device_id=(src,),
                device_id_type=pl.DeviceIdType.MESH,
            ).wait_recv()
        for s in range(N_DEV):

            @pl.when(s != my)
            def _(s=s):
                pltpu.make_async_remote_copy(
                    src_ref=zbuf.at[s],
                    dst_ref=recv_buf.at[my],
                    send_sem=send_sems.at[s],
                    recv_sem=recv_sems.at[my],
                    device_id=((my + 1) % N_DEV,),
                    device_id_type=pl.DeviceIdType.MESH,
                ).wait_send()

        out_ref[...] = recv_buf[...].astype(jnp.float32).reshape(
            N_DEV * m_per, n_per)

    return pl.pallas_call(
        body,
        out_shape=jax.ShapeDtypeStruct((N_DEV * m_per, n_per), jnp.float32),
        in_specs=[
            pl.BlockSpec(memory_space=pltpu.VMEM),
            pl.BlockSpec(memory_space=pltpu.MemorySpace.HBM),
            pl.BlockSpec(memory_space=pltpu.SMEM),
            pl.BlockSpec(memory_space=pltpu.SMEM),
        ],
        out_specs=pl.BlockSpec(memory_space=pltpu.VMEM),
        scratch_shapes=[
            pltpu.VMEM((SLOTS, k // KC, n // NQ), jnp.float32),
            pltpu.VMEM((N_DEV, m_per, n // N_DEV), jnp.bfloat16),
            pltpu.VMEM((N_DEV, m_per, n // N_DEV), jnp.bfloat16),
            pltpu.SemaphoreType.DMA((SLOTS,)),
            pltpu.SemaphoreType.DMA((N_DEV,)),
            pltpu.SemaphoreType.DMA((N_DEV,)),
            pltpu.SemaphoreType.DMA,
        ],
        compiler_params=pltpu.CompilerParams(
            collective_id=0,
            vmem_limit_bytes=128 * 1024 * 1024,
        ),
    )(x, w_mat, scale_x, scale_w)
